# baseline (speedup 1.0000x reference)
import functools

import jax
import jax.numpy as jnp
from jax import lax
from jax.experimental import pallas as pl
from jax.experimental.pallas import tpu as pltpu
from jax.experimental.pallas import tpu_sc as plsc

_THRESH = 0.7
_LOG_THRESH = float(__import__("math").log(0.7))
_MIN_KEPT = 131072

_NC = 2
_NS = 16
_NL = 16
_NTILES = _NC * _NS
_NBINS = 256


def _stats_body(pred_ref, targ_ref, thr_ref, cnt_ref, sum_ref, *, c, log_space):
    t = targ_ref[0]
    s = None
    picked = pred_ref[0, 0]
    for ci in range(c):
        xi = pred_ref[0, ci]
        e = jnp.exp(xi)
        s = e if s is None else s + e
        if ci > 0:
            picked = jnp.where(t == ci, xi, picked)
    logp = picked - jnp.log(s)
    if log_space:
        keep = logp <= thr_ref[0, 0]
    else:
        keep = jnp.exp(logp) <= thr_ref[0, 0]
    cnt_blk = jnp.sum(keep.astype(jnp.float32))
    sum_blk = jnp.sum(jnp.where(keep, -logp, 0.0))

    @pl.when(pl.program_id(0) == 0)
    def _():
        cnt_ref[0, 0] = 0.0
        sum_ref[0, 0] = 0.0

    cnt_ref[0, 0] += cnt_blk
    sum_ref[0, 0] += sum_blk


def _stats(pred, target, thr, log_space, hb=256, rows=None):
    b, c, h, w = pred.shape
    if rows is None:
        rows = h
    if rows % hb:
        hb = rows
    nh = rows // hb
    cnt, ssum = pl.pallas_call(
        functools.partial(_stats_body, c=c, log_space=log_space),
        grid=(b * nh,),
        in_specs=[
            pl.BlockSpec((1, c, hb, w), lambda i: (i // nh, 0, i % nh, 0)),
            pl.BlockSpec((1, hb, w), lambda i: (i // nh, i % nh, 0)),
            pl.BlockSpec(memory_space=pltpu.SMEM),
        ],
        out_specs=[
            pl.BlockSpec(memory_space=pltpu.SMEM),
            pl.BlockSpec(memory_space=pltpu.SMEM),
        ],
        out_shape=[
            jax.ShapeDtypeStruct((1, 1), jnp.float32),
            jax.ShapeDtypeStruct((1, 1), jnp.float32),
        ],
    )(pred, target, thr)
    return cnt[0, 0], ssum[0, 0]


def _p_body(pred_ref, targ_ref, p_ref, *, c):
    t = targ_ref[0]
    s = None
    picked = pred_ref[0, 0]
    for ci in range(c):
        xi = pred_ref[0, ci]
        e = jnp.exp(xi)
        s = e if s is None else s + e
        if ci > 0:
            picked = jnp.where(t == ci, xi, picked)
    logp = picked - jnp.log(s)
    p_ref[0] = jnp.exp(logp)


def _compute_p(pred, target, hb=256):
    b, c, h, w = pred.shape
    if h % hb:
        hb = h
    nh = h // hb
    return pl.pallas_call(
        functools.partial(_p_body, c=c),
        grid=(b * nh,),
        in_specs=[
            pl.BlockSpec((1, c, hb, w), lambda i: (i // nh, 0, i % nh, 0)),
            pl.BlockSpec((1, hb, w), lambda i: (i // nh, i % nh, 0)),
        ],
        out_specs=pl.BlockSpec((1, hb, w), lambda i: (i // nh, i % nh, 0)),
        out_shape=jax.ShapeDtypeStruct((b, h, w), jnp.float32),
    )(pred, target)


def _sc_hist(pf, pref16, pass_i):
    n = pf.shape[0]
    chunk = n // _NTILES
    mesh = plsc.VectorSubcoreMesh(
        core_axis_name="c", subcore_axis_name="s",
        num_cores=_NC, num_subcores=_NS,
    )

    @functools.partial(
        pl.kernel,
        mesh=mesh,
        out_type=jax.ShapeDtypeStruct((_NTILES * _NBINS,), jnp.int32),
        compiler_params=pltpu.CompilerParams(needs_layout_passes=False),
        scratch_types=[
            pltpu.VMEM((chunk,), jnp.int32),
            pltpu.VMEM((_NBINS,), jnp.int32),
            pltpu.VMEM((_NL,), jnp.int32),
        ],
    )
    def k(p_hbm, pref_hbm, out_hbm, data_v, hist_v, pref_v):
        wid = lax.axis_index("c") * _NS + lax.axis_index("s")
        pltpu.sync_copy(p_hbm.at[pl.ds(wid * chunk, chunk)], data_v)
        pltpu.sync_copy(pref_hbm, pref_v)
        for j in range(_NBINS // _NL):
            hist_v[pl.ds(j * _NL, _NL)] = jnp.zeros((_NL,), jnp.int32)
        prefv = pref_v[...]
        ones = jnp.ones((_NL,), jnp.int32)

        def body(i, carry):
            bits = data_v[pl.ds(i * _NL, _NL)]
            bin_ = lax.shift_right_logical(bits, 24 - 8 * pass_i) & 255
            if pass_i == 0:
                mask = bits == bits
            else:
                mask = lax.shift_right_logical(bits, 32 - 8 * pass_i) == prefv
            plsc.addupdate_scatter(hist_v, [bin_], ones, mask=mask)
            return carry

        lax.fori_loop(0, chunk // _NL, body, 0)
        pltpu.sync_copy(hist_v, out_hbm.at[pl.ds(wid * _NBINS, _NBINS)])

    return k(pf, pref16)


def _hist_reduce_body(hist_ref, out_ref):
    out_ref[...] = jnp.sum(hist_ref[...], axis=0, keepdims=True)


def _hist_reduce(hist):
    return pl.pallas_call(
        _hist_reduce_body,
        out_shape=jax.ShapeDtypeStruct((1, _NBINS), jnp.int32),
    )(hist.reshape(_NTILES, _NBINS))


def _kth_smallest_sc(pf, rank0):
    bits = lax.bitcast_convert_type(pf, jnp.int32)
    pref = jnp.zeros((), jnp.int32)
    rank = jnp.asarray(rank0, jnp.int32)
    for pass_i in range(4):
        pref16 = jnp.full((_NL,), pref, jnp.int32)
        hist = _sc_hist(bits, pref16, pass_i)
        hsum = _hist_reduce(hist)[0]
        cum = jnp.cumsum(hsum)
        digit = jnp.argmax(cum > rank).astype(jnp.int32)
        below = cum[digit] - hsum[digit]
        rank = rank - below
        pref = pref * 256 + digit
    return lax.bitcast_convert_type(pref, jnp.float32)


def _sc_softmax(pred, target, rows_tc):
    b, c, h, w = pred.shape
    sc_rows = h - rows_tc
    tiles_per_b = _NTILES // b
    rows_tile = sc_rows // tiles_per_b
    blk_rows = 4
    blk = blk_rows * w
    rounds = rows_tile // blk_rows
    n_sc = b * sc_rows * w
    pf = pred.reshape(-1)
    tf = target.reshape(-1)
    mesh = plsc.VectorSubcoreMesh(
        core_axis_name="c", subcore_axis_name="s",
        num_cores=_NC, num_subcores=_NS,
    )

    @functools.partial(
        pl.kernel,
        mesh=mesh,
        out_type=[
            jax.ShapeDtypeStruct((n_sc,), jnp.float32),
            jax.ShapeDtypeStruct((_NTILES * 2 * _NL,), jnp.float32),
        ],
        compiler_params=pltpu.CompilerParams(needs_layout_passes=False),
        scratch_types=[
            pltpu.VMEM((c * blk,), jnp.float32),
            pltpu.VMEM((c * blk,), jnp.float32),
            pltpu.VMEM((blk,), jnp.int32),
            pltpu.VMEM((blk,), jnp.int32),
            pltpu.VMEM((blk,), jnp.float32),
            pltpu.VMEM((blk,), jnp.float32),
            pltpu.VMEM((2 * _NL,), jnp.float32),
            pltpu.SemaphoreType.DMA,
            pltpu.SemaphoreType.DMA,
            pltpu.SemaphoreType.DMA,
            pltpu.SemaphoreType.DMA,
        ],
    )
    def k(p_hbm, t_hbm, sout_hbm, acc_hbm, bufA, bufB, tbufA, tbufB,
          obufA, obufB, accv, semA, semB, osemA, osemB):
        wid = lax.axis_index("c") * _NS + lax.axis_index("s")
        bi = wid // tiles_per_b
        qi = wid % tiles_per_b
        row0_tile = rows_tc + qi * rows_tile
        iota = lax.broadcasted_iota(jnp.int32, (_NL,), 0)
        bufs = (bufA, bufB)
        tbufs = (tbufA, tbufB)
        obufs = (obufA, obufB)
        sems = (semA, semB)
        osems = (osemA, osemB)

        def start_round(r):
            par = r % 2
            row0 = row0_tile + r * blk_rows
            hnd = []
            for ci in range(c):
                off = ((bi * c + ci) * h + row0) * w
                hnd.append(pltpu.async_copy(
                    p_hbm.at[pl.ds(off, blk)],
                    bufs[par].at[pl.ds(ci * blk, blk)], sems[par]))
            toff = (bi * h + row0) * w
            hnd.append(pltpu.async_copy(
                t_hbm.at[pl.ds(toff, blk)], tbufs[par], sems[par]))
            return hnd

        pend = {0: start_round(0)}
        opend = {}
        cnt_c = jnp.zeros((_NL,), jnp.float32)
        sp_c = jnp.zeros((_NL,), jnp.float32)
        for r in range(rounds):
            par = r % 2
            if r + 1 < rounds:
                pend[r + 1] = start_round(r + 1)
            for hdl in pend.pop(r):
                hdl.wait()
            buf, tbuf, obuf = bufs[par], tbufs[par], obufs[par]
            if r - 2 in opend:
                opend.pop(r - 2).wait()

            def body(i, carry):
                cnt_a, sp_a = carry
                t_vec = tbuf[pl.ds(i * _NL, _NL)]
                sv = None
                for ci in range(c):
                    v = buf[pl.ds(ci * blk + i * _NL, _NL)]
                    e = jnp.exp(v)
                    sv = e if sv is None else sv + e
                gidx = t_vec * blk + i * _NL + iota
                picked = plsc.load_gather(buf, [gidx])
                p_val = jnp.exp(picked) / sv
                keep = p_val <= jnp.float32(_THRESH)
                obuf[pl.ds(i * _NL, _NL)] = jnp.where(keep, sv, 1.0)
                cnt_a = cnt_a + jnp.where(keep, 1.0, 0.0)
                sp_a = sp_a + jnp.where(keep, picked, 0.0)
                return (cnt_a, sp_a)

            cnt_c, sp_c = lax.fori_loop(0, blk // _NL, body, (cnt_c, sp_c))
            oidx = (bi * sc_rows + qi * rows_tile + r * blk_rows) * w
            opend[r] = pltpu.async_copy(
                obuf, sout_hbm.at[pl.ds(oidx, blk)], osems[par])
        for r in list(opend):
            opend.pop(r).wait()
        accv[pl.ds(0, _NL)] = cnt_c
        accv[pl.ds(_NL, _NL)] = sp_c
        pltpu.sync_copy(accv, acc_hbm.at[pl.ds(wid * 2 * _NL, 2 * _NL)])

    return k(pf, tf)


def _logsum_body(s_ref, out_ref):
    blksum = jnp.sum(jnp.log(s_ref[...]))

    @pl.when(pl.program_id(0) == 0)
    def _():
        out_ref[0, 0] = 0.0

    out_ref[0, 0] += blksum


def _logsum(sprime):
    n = sprime.shape[0]
    rows = 1024
    cols = n // rows
    grid = 2
    return pl.pallas_call(
        _logsum_body,
        grid=(grid,),
        in_specs=[pl.BlockSpec((rows // grid, cols), lambda i: (i, 0))],
        out_specs=pl.BlockSpec(memory_space=pltpu.SMEM),
        out_shape=jax.ShapeDtypeStruct((1, 1), jnp.float32),
    )(sprime.reshape(rows, cols))[0, 0]


def kernel(pred, target):
    b, c, h, w = pred.shape
    n = b * h * w
    k = min(n, _MIN_KEPT)
    thr0 = jnp.full((1, 1), _LOG_THRESH, jnp.float32)
    hybrid_ok = (
        b <= _NTILES and _NTILES % b == 0
        and (h // 2) % 4 == 0
        and ((h - h // 2) % (_NTILES // b)) == 0
        and (((h - h // 2) // (_NTILES // b)) % 4 == 0)
        and (4 * w) % _NL == 0
    )
    if hybrid_ok:
        rows_tc = h // 2
        cnt_tc, sum_tc = _stats(pred, target, thr0, log_space=True,
                                rows=rows_tc)
        sprime, accs = _sc_softmax(pred, target, rows_tc)
        ls = _logsum(sprime)
        acc2 = accs.reshape(_NTILES, 2, _NL)
        cnt_sc = jnp.sum(acc2[:, 0, :])
        sp_sc = jnp.sum(acc2[:, 1, :])
        cnt0 = cnt_tc + cnt_sc
        sum0 = sum_tc + ls - sp_sc
    else:
        cnt0, sum0 = _stats(pred, target, thr0, log_space=True)

    def fast():
        return sum0 / jnp.maximum(cnt0, 1.0)

    def slow():
        p = _compute_p(pred, target).reshape(-1)
        kth = _kth_smallest_sc(p, k - 1)
        thr = jnp.maximum(kth, jnp.float32(_THRESH)).reshape(1, 1)
        cnt, ssum = _stats(pred, target, thr, log_space=False)
        return ssum / jnp.maximum(cnt, 1.0)

    return lax.cond(cnt0 >= jnp.float32(k), fast, slow)


# speedup vs baseline: 1.7865x; 1.7865x over previous
import functools

import jax
import jax.numpy as jnp
from jax import lax
from jax.experimental import pallas as pl
from jax.experimental.pallas import tpu as pltpu
from jax.experimental.pallas import tpu_sc as plsc

_THRESH = 0.7
_LOG_THRESH = float(__import__("math").log(0.7))
_MIN_KEPT = 131072

_NC = 2
_NS = 16
_NL = 16
_NTILES = _NC * _NS
_NBINS = 256


def _stats_body(pred_ref, targ_ref, thr_ref, cnt_ref, sum_ref, *, c, log_space):
    t = targ_ref[0]
    s = None
    picked = pred_ref[0, 0]
    for ci in range(c):
        xi = pred_ref[0, ci]
        e = jnp.exp(xi)
        s = e if s is None else s + e
        if ci > 0:
            picked = jnp.where(t == ci, xi, picked)
    logp = picked - jnp.log(s)
    if log_space:
        keep = logp <= thr_ref[0, 0]
    else:
        keep = jnp.exp(logp) <= thr_ref[0, 0]
    cnt_blk = jnp.sum(keep.astype(jnp.float32))
    sum_blk = jnp.sum(jnp.where(keep, -logp, 0.0))

    @pl.when(pl.program_id(0) == 0)
    def _():
        cnt_ref[0, 0] = 0.0
        sum_ref[0, 0] = 0.0

    cnt_ref[0, 0] += cnt_blk
    sum_ref[0, 0] += sum_blk


def _stats(pred, target, thr, log_space, hb=256, rows=None):
    b, c, h, w = pred.shape
    if rows is None:
        rows = h
    if rows % hb:
        hb = rows
    nh = rows // hb
    cnt, ssum = pl.pallas_call(
        functools.partial(_stats_body, c=c, log_space=log_space),
        grid=(b * nh,),
        in_specs=[
            pl.BlockSpec((1, c, hb, w), lambda i: (i // nh, 0, i % nh, 0)),
            pl.BlockSpec((1, hb, w), lambda i: (i // nh, i % nh, 0)),
            pl.BlockSpec(memory_space=pltpu.SMEM),
        ],
        out_specs=[
            pl.BlockSpec(memory_space=pltpu.SMEM),
            pl.BlockSpec(memory_space=pltpu.SMEM),
        ],
        out_shape=[
            jax.ShapeDtypeStruct((1, 1), jnp.float32),
            jax.ShapeDtypeStruct((1, 1), jnp.float32),
        ],
    )(pred, target, thr)
    return cnt[0, 0], ssum[0, 0]


def _p_body(pred_ref, targ_ref, p_ref, *, c):
    t = targ_ref[0]
    s = None
    picked = pred_ref[0, 0]
    for ci in range(c):
        xi = pred_ref[0, ci]
        e = jnp.exp(xi)
        s = e if s is None else s + e
        if ci > 0:
            picked = jnp.where(t == ci, xi, picked)
    logp = picked - jnp.log(s)
    p_ref[0] = jnp.exp(logp)


def _compute_p(pred, target, hb=256):
    b, c, h, w = pred.shape
    if h % hb:
        hb = h
    nh = h // hb
    return pl.pallas_call(
        functools.partial(_p_body, c=c),
        grid=(b * nh,),
        in_specs=[
            pl.BlockSpec((1, c, hb, w), lambda i: (i // nh, 0, i % nh, 0)),
            pl.BlockSpec((1, hb, w), lambda i: (i // nh, i % nh, 0)),
        ],
        out_specs=pl.BlockSpec((1, hb, w), lambda i: (i // nh, i % nh, 0)),
        out_shape=jax.ShapeDtypeStruct((b, h, w), jnp.float32),
    )(pred, target)


def _sc_hist(pf, pref16, pass_i):
    n = pf.shape[0]
    chunk = n // _NTILES
    mesh = plsc.VectorSubcoreMesh(
        core_axis_name="c", subcore_axis_name="s",
        num_cores=_NC, num_subcores=_NS,
    )

    @functools.partial(
        pl.kernel,
        mesh=mesh,
        out_type=jax.ShapeDtypeStruct((_NTILES * _NBINS,), jnp.int32),
        compiler_params=pltpu.CompilerParams(needs_layout_passes=False),
        scratch_types=[
            pltpu.VMEM((chunk,), jnp.int32),
            pltpu.VMEM((_NBINS,), jnp.int32),
            pltpu.VMEM((_NL,), jnp.int32),
        ],
    )
    def k(p_hbm, pref_hbm, out_hbm, data_v, hist_v, pref_v):
        wid = lax.axis_index("c") * _NS + lax.axis_index("s")
        pltpu.sync_copy(p_hbm.at[pl.ds(wid * chunk, chunk)], data_v)
        pltpu.sync_copy(pref_hbm, pref_v)
        for j in range(_NBINS // _NL):
            hist_v[pl.ds(j * _NL, _NL)] = jnp.zeros((_NL,), jnp.int32)
        prefv = pref_v[...]
        ones = jnp.ones((_NL,), jnp.int32)

        def body(i, carry):
            bits = data_v[pl.ds(i * _NL, _NL)]
            bin_ = lax.shift_right_logical(bits, 24 - 8 * pass_i) & 255
            if pass_i == 0:
                mask = bits == bits
            else:
                mask = lax.shift_right_logical(bits, 32 - 8 * pass_i) == prefv
            plsc.addupdate_scatter(hist_v, [bin_], ones, mask=mask)
            return carry

        lax.fori_loop(0, chunk // _NL, body, 0)
        pltpu.sync_copy(hist_v, out_hbm.at[pl.ds(wid * _NBINS, _NBINS)])

    return k(pf, pref16)


def _hist_reduce_body(hist_ref, out_ref):
    out_ref[...] = jnp.sum(hist_ref[...], axis=0, keepdims=True)


def _hist_reduce(hist):
    return pl.pallas_call(
        _hist_reduce_body,
        out_shape=jax.ShapeDtypeStruct((1, _NBINS), jnp.int32),
    )(hist.reshape(_NTILES, _NBINS))


def _kth_smallest_sc(pf, rank0):
    bits = lax.bitcast_convert_type(pf, jnp.int32)
    pref = jnp.zeros((), jnp.int32)
    rank = jnp.asarray(rank0, jnp.int32)
    for pass_i in range(4):
        pref16 = jnp.full((_NL,), pref, jnp.int32)
        hist = _sc_hist(bits, pref16, pass_i)
        hsum = _hist_reduce(hist)[0]
        cum = jnp.cumsum(hsum)
        digit = jnp.argmax(cum > rank).astype(jnp.int32)
        below = cum[digit] - hsum[digit]
        rank = rank - below
        pref = pref * 256 + digit
    return lax.bitcast_convert_type(pref, jnp.float32)


def _sc_softmax(pred, target, rows_tc):
    b, c, h, w = pred.shape
    sc_rows = h - rows_tc
    tiles_per_b = _NTILES // b
    rows_tile = sc_rows // tiles_per_b
    blk_rows = 4
    blk = blk_rows * w
    rounds = rows_tile // blk_rows
    n_sc = b * sc_rows * w
    tf = target.reshape(-1)
    mesh = plsc.VectorSubcoreMesh(
        core_axis_name="c", subcore_axis_name="s",
        num_cores=_NC, num_subcores=_NS,
    )
    cpr = w // _NL

    @functools.partial(
        pl.kernel,
        mesh=mesh,
        out_type=[
            jax.ShapeDtypeStruct((n_sc,), jnp.float32),
            jax.ShapeDtypeStruct((_NTILES * 2 * _NL,), jnp.float32),
        ],
        compiler_params=pltpu.CompilerParams(needs_layout_passes=False),
        scratch_types=[
            pltpu.VMEM((c, blk_rows, w), jnp.float32),
            pltpu.VMEM((c, blk_rows, w), jnp.float32),
            pltpu.VMEM((blk,), jnp.int32),
            pltpu.VMEM((blk,), jnp.int32),
            pltpu.VMEM((blk,), jnp.float32),
            pltpu.VMEM((blk,), jnp.float32),
            pltpu.VMEM((2 * _NL,), jnp.float32),
            pltpu.SemaphoreType.DMA,
            pltpu.SemaphoreType.DMA,
            pltpu.SemaphoreType.DMA,
            pltpu.SemaphoreType.DMA,
        ],
    )
    def k(p_hbm, t_hbm, sout_hbm, acc_hbm, bufA, bufB, tbufA, tbufB,
          obufA, obufB, accv, semA, semB, osemA, osemB):
        wid = lax.axis_index("c") * _NS + lax.axis_index("s")
        bi = wid // tiles_per_b
        qi = wid % tiles_per_b
        row0_tile = rows_tc + qi * rows_tile
        iota = lax.broadcasted_iota(jnp.int32, (_NL,), 0)
        bufs = (bufA, bufB)
        tbufs = (tbufA, tbufB)
        obufs = (obufA, obufB)
        sems = (semA, semB)
        osems = (osemA, osemB)

        def start_round(r):
            par = r % 2
            row0 = row0_tile + r * blk_rows
            hnd = [pltpu.async_copy(
                p_hbm.at[bi, :, pl.ds(row0, blk_rows), :],
                bufs[par], sems[par])]
            toff = (bi * h + row0) * w
            hnd.append(pltpu.async_copy(
                t_hbm.at[pl.ds(toff, blk)], tbufs[par], sems[par]))
            return hnd

        pend = {0: start_round(0)}
        opend = {}
        cnt_c = jnp.zeros((_NL,), jnp.float32)
        sp_c = jnp.zeros((_NL,), jnp.float32)
        for r in range(rounds):
            par = r % 2
            if r + 1 < rounds:
                pend[r + 1] = start_round(r + 1)
            for hdl in pend.pop(r):
                hdl.wait()
            buf, tbuf, obuf = bufs[par], tbufs[par], obufs[par]
            if r - 2 in opend:
                opend.pop(r - 2).wait()

            def body(j, carry):
                cnt_a, sp_a = carry
                col = j * _NL
                for rr in range(blk_rows):
                    t_vec = tbuf[pl.ds(rr * w + col, _NL)]
                    sv = None
                    for ci in range(c):
                        e = jnp.exp(buf[ci, rr, pl.ds(col, _NL)])
                        sv = e if sv is None else sv + e
                    rr_vec = jnp.full((_NL,), rr, jnp.int32)
                    cidx = col + iota
                    picked = plsc.load_gather(buf, [t_vec, rr_vec, cidx])
                    p_val = jnp.exp(picked) / sv
                    keep = p_val <= jnp.float32(_THRESH)
                    obuf[pl.ds(rr * w + col, _NL)] = jnp.where(keep, sv, 1.0)
                    cnt_a = cnt_a + jnp.where(keep, 1.0, 0.0)
                    sp_a = sp_a + jnp.where(keep, picked, 0.0)
                return (cnt_a, sp_a)

            cnt_c, sp_c = lax.fori_loop(0, cpr, body, (cnt_c, sp_c))
            oidx = (bi * sc_rows + qi * rows_tile + r * blk_rows) * w
            opend[r] = pltpu.async_copy(
                obuf, sout_hbm.at[pl.ds(oidx, blk)], osems[par])
        for r in list(opend):
            opend.pop(r).wait()
        accv[pl.ds(0, _NL)] = cnt_c
        accv[pl.ds(_NL, _NL)] = sp_c
        pltpu.sync_copy(accv, acc_hbm.at[pl.ds(wid * 2 * _NL, 2 * _NL)])

    return k(pred, tf)


def _logsum_body(s_ref, out_ref):
    blksum = jnp.sum(jnp.log(s_ref[...]))

    @pl.when(pl.program_id(0) == 0)
    def _():
        out_ref[0, 0] = 0.0

    out_ref[0, 0] += blksum


def _logsum(sprime):
    n = sprime.shape[0]
    rows = 1024
    cols = n // rows
    grid = 2
    return pl.pallas_call(
        _logsum_body,
        grid=(grid,),
        in_specs=[pl.BlockSpec((rows // grid, cols), lambda i: (i, 0))],
        out_specs=pl.BlockSpec(memory_space=pltpu.SMEM),
        out_shape=jax.ShapeDtypeStruct((1, 1), jnp.float32),
    )(sprime.reshape(rows, cols))[0, 0]


def kernel(pred, target):
    b, c, h, w = pred.shape
    n = b * h * w
    k = min(n, _MIN_KEPT)
    thr0 = jnp.full((1, 1), _LOG_THRESH, jnp.float32)
    hybrid_ok = (
        b <= _NTILES and _NTILES % b == 0
        and (h // 2) % 4 == 0
        and ((h - h // 2) % (_NTILES // b)) == 0
        and (((h - h // 2) // (_NTILES // b)) % 4 == 0)
        and (4 * w) % _NL == 0
    )
    if hybrid_ok:
        rows_tc = h // 2
        cnt_tc, sum_tc = _stats(pred, target, thr0, log_space=True,
                                rows=rows_tc)
        sprime, accs = _sc_softmax(pred, target, rows_tc)
        ls = _logsum(sprime)
        acc2 = accs.reshape(_NTILES, 2, _NL)
        cnt_sc = jnp.sum(acc2[:, 0, :])
        sp_sc = jnp.sum(acc2[:, 1, :])
        cnt0 = cnt_tc + cnt_sc
        sum0 = sum_tc + ls - sp_sc
    else:
        cnt0, sum0 = _stats(pred, target, thr0, log_space=True)

    def fast():
        return sum0 / jnp.maximum(cnt0, 1.0)

    def slow():
        p = _compute_p(pred, target).reshape(-1)
        kth = _kth_smallest_sc(p, k - 1)
        thr = jnp.maximum(kth, jnp.float32(_THRESH)).reshape(1, 1)
        cnt, ssum = _stats(pred, target, thr, log_space=False)
        return ssum / jnp.maximum(cnt, 1.0)

    return lax.cond(cnt0 >= jnp.float32(k), fast, slow)


# speedup vs baseline: 2.7160x; 1.5203x over previous
import functools

import jax
import jax.numpy as jnp
from jax import lax
from jax.experimental import pallas as pl
from jax.experimental.pallas import tpu as pltpu
from jax.experimental.pallas import tpu_sc as plsc

_THRESH = 0.7
_LOG_THRESH = float(__import__("math").log(0.7))
_MIN_KEPT = 131072

_NC = 2
_NS = 16
_NL = 16
_NTILES = _NC * _NS
_NBINS = 256


def _stats_body(pred_ref, targ_ref, thr_ref, cnt_ref, sum_ref, *, c, log_space):
    t = targ_ref[0]
    s = None
    picked = pred_ref[0, 0]
    for ci in range(c):
        xi = pred_ref[0, ci]
        e = jnp.exp(xi)
        s = e if s is None else s + e
        if ci > 0:
            picked = jnp.where(t == ci, xi, picked)
    logp = picked - jnp.log(s)
    if log_space:
        keep = logp <= thr_ref[0, 0]
    else:
        keep = jnp.exp(logp) <= thr_ref[0, 0]
    cnt_blk = jnp.sum(keep.astype(jnp.float32))
    sum_blk = jnp.sum(jnp.where(keep, -logp, 0.0))

    @pl.when(pl.program_id(0) == 0)
    def _():
        cnt_ref[0, 0] = 0.0
        sum_ref[0, 0] = 0.0

    cnt_ref[0, 0] += cnt_blk
    sum_ref[0, 0] += sum_blk


def _stats(pred, target, thr, log_space, hb=256, rows=None):
    b, c, h, w = pred.shape
    if rows is None:
        rows = h
    if rows % hb:
        hb = rows
    nh = rows // hb
    cnt, ssum = pl.pallas_call(
        functools.partial(_stats_body, c=c, log_space=log_space),
        grid=(b * nh,),
        in_specs=[
            pl.BlockSpec((1, c, hb, w), lambda i: (i // nh, 0, i % nh, 0)),
            pl.BlockSpec((1, hb, w), lambda i: (i // nh, i % nh, 0)),
            pl.BlockSpec(memory_space=pltpu.SMEM),
        ],
        out_specs=[
            pl.BlockSpec(memory_space=pltpu.SMEM),
            pl.BlockSpec(memory_space=pltpu.SMEM),
        ],
        out_shape=[
            jax.ShapeDtypeStruct((1, 1), jnp.float32),
            jax.ShapeDtypeStruct((1, 1), jnp.float32),
        ],
    )(pred, target, thr)
    return cnt[0, 0], ssum[0, 0]


def _p_body(pred_ref, targ_ref, p_ref, *, c):
    t = targ_ref[0]
    s = None
    picked = pred_ref[0, 0]
    for ci in range(c):
        xi = pred_ref[0, ci]
        e = jnp.exp(xi)
        s = e if s is None else s + e
        if ci > 0:
            picked = jnp.where(t == ci, xi, picked)
    logp = picked - jnp.log(s)
    p_ref[0] = jnp.exp(logp)


def _compute_p(pred, target, hb=256):
    b, c, h, w = pred.shape
    if h % hb:
        hb = h
    nh = h // hb
    return pl.pallas_call(
        functools.partial(_p_body, c=c),
        grid=(b * nh,),
        in_specs=[
            pl.BlockSpec((1, c, hb, w), lambda i: (i // nh, 0, i % nh, 0)),
            pl.BlockSpec((1, hb, w), lambda i: (i // nh, i % nh, 0)),
        ],
        out_specs=pl.BlockSpec((1, hb, w), lambda i: (i // nh, i % nh, 0)),
        out_shape=jax.ShapeDtypeStruct((b, h, w), jnp.float32),
    )(pred, target)


def _sc_hist(pf, pref16, pass_i):
    n = pf.shape[0]
    chunk = n // _NTILES
    mesh = plsc.VectorSubcoreMesh(
        core_axis_name="c", subcore_axis_name="s",
        num_cores=_NC, num_subcores=_NS,
    )

    @functools.partial(
        pl.kernel,
        mesh=mesh,
        out_type=jax.ShapeDtypeStruct((_NTILES * _NBINS,), jnp.int32),
        compiler_params=pltpu.CompilerParams(needs_layout_passes=False),
        scratch_types=[
            pltpu.VMEM((chunk,), jnp.int32),
            pltpu.VMEM((_NBINS,), jnp.int32),
            pltpu.VMEM((_NL,), jnp.int32),
        ],
    )
    def k(p_hbm, pref_hbm, out_hbm, data_v, hist_v, pref_v):
        wid = lax.axis_index("c") * _NS + lax.axis_index("s")
        pltpu.sync_copy(p_hbm.at[pl.ds(wid * chunk, chunk)], data_v)
        pltpu.sync_copy(pref_hbm, pref_v)
        for j in range(_NBINS // _NL):
            hist_v[pl.ds(j * _NL, _NL)] = jnp.zeros((_NL,), jnp.int32)
        prefv = pref_v[...]
        ones = jnp.ones((_NL,), jnp.int32)

        def body(i, carry):
            bits = data_v[pl.ds(i * _NL, _NL)]
            bin_ = lax.shift_right_logical(bits, 24 - 8 * pass_i) & 255
            if pass_i == 0:
                mask = bits == bits
            else:
                mask = lax.shift_right_logical(bits, 32 - 8 * pass_i) == prefv
            plsc.addupdate_scatter(hist_v, [bin_], ones, mask=mask)
            return carry

        lax.fori_loop(0, chunk // _NL, body, 0)
        pltpu.sync_copy(hist_v, out_hbm.at[pl.ds(wid * _NBINS, _NBINS)])

    return k(pf, pref16)


def _hist_reduce_body(hist_ref, out_ref):
    out_ref[...] = jnp.sum(hist_ref[...], axis=0, keepdims=True)


def _hist_reduce(hist):
    return pl.pallas_call(
        _hist_reduce_body,
        out_shape=jax.ShapeDtypeStruct((1, _NBINS), jnp.int32),
    )(hist.reshape(_NTILES, _NBINS))


def _kth_smallest_sc(pf, rank0):
    bits = lax.bitcast_convert_type(pf, jnp.int32)
    pref = jnp.zeros((), jnp.int32)
    rank = jnp.asarray(rank0, jnp.int32)
    for pass_i in range(4):
        pref16 = jnp.full((_NL,), pref, jnp.int32)
        hist = _sc_hist(bits, pref16, pass_i)
        hsum = _hist_reduce(hist)[0]
        cum = jnp.cumsum(hsum)
        digit = jnp.argmax(cum > rank).astype(jnp.int32)
        below = cum[digit] - hsum[digit]
        rank = rank - below
        pref = pref * 256 + digit
    return lax.bitcast_convert_type(pref, jnp.float32)


def _sc_softmax(pred, target, rows_tc):
    b, c, h, w = pred.shape
    sc_rows = h - rows_tc
    tiles_per_b = _NTILES // b
    rows_tile = sc_rows // tiles_per_b
    blk_rows = 4
    blk = blk_rows * w
    rounds = rows_tile // blk_rows
    n_sc = b * sc_rows * w
    tf = target.reshape(-1)
    mesh = plsc.VectorSubcoreMesh(
        core_axis_name="c", subcore_axis_name="s",
        num_cores=_NC, num_subcores=_NS,
    )
    cpr = w // _NL

    @functools.partial(
        pl.kernel,
        mesh=mesh,
        out_type=[
            jax.ShapeDtypeStruct((n_sc,), jnp.float32),
            jax.ShapeDtypeStruct((_NTILES * 2 * _NL,), jnp.float32),
        ],
        compiler_params=pltpu.CompilerParams(needs_layout_passes=False),
        scratch_types=[
            pltpu.VMEM((c, blk_rows, w), jnp.float32),
            pltpu.VMEM((c, blk_rows, w), jnp.float32),
            pltpu.VMEM((blk,), jnp.int32),
            pltpu.VMEM((blk,), jnp.int32),
            pltpu.VMEM((blk,), jnp.float32),
            pltpu.VMEM((blk,), jnp.float32),
            pltpu.VMEM((2 * _NL,), jnp.float32),
            pltpu.SemaphoreType.DMA,
            pltpu.SemaphoreType.DMA,
            pltpu.SemaphoreType.DMA,
            pltpu.SemaphoreType.DMA,
        ],
    )
    def k(p_hbm, t_hbm, sout_hbm, acc_hbm, bufA, bufB, tbufA, tbufB,
          obufA, obufB, accv, semA, semB, osemA, osemB):
        wid = lax.axis_index("c") * _NS + lax.axis_index("s")
        bi = wid // tiles_per_b
        qi = wid % tiles_per_b
        row0_tile = rows_tc + qi * rows_tile
        iota = lax.broadcasted_iota(jnp.int32, (_NL,), 0)
        bufs = (bufA, bufB)
        tbufs = (tbufA, tbufB)
        obufs = (obufA, obufB)
        sems = (semA, semB)
        osems = (osemA, osemB)

        def start_round(r):
            par = r % 2
            row0 = row0_tile + r * blk_rows
            hnd = [pltpu.async_copy(
                p_hbm.at[bi, :, pl.ds(row0, blk_rows), :],
                bufs[par], sems[par])]
            toff = (bi * h + row0) * w
            hnd.append(pltpu.async_copy(
                t_hbm.at[pl.ds(toff, blk)], tbufs[par], sems[par]))
            return hnd

        pend = {0: start_round(0)}
        opend = {}
        cnt_c = jnp.zeros((_NL,), jnp.float32)
        sp_c = jnp.zeros((_NL,), jnp.float32)
        for r in range(rounds):
            par = r % 2
            if r + 1 < rounds:
                pend[r + 1] = start_round(r + 1)
            for hdl in pend.pop(r):
                hdl.wait()
            buf, tbuf, obuf = bufs[par], tbufs[par], obufs[par]
            if r - 2 in opend:
                opend.pop(r - 2).wait()

            def body(j, carry):
                cnt_a, sp_a = carry
                col = j * _NL
                for rr in range(blk_rows):
                    t_vec = tbuf[pl.ds(rr * w + col, _NL)]
                    sv = None
                    for ci in range(c):
                        e = jnp.exp(buf[ci, rr, pl.ds(col, _NL)])
                        sv = e if sv is None else sv + e
                    rr_vec = jnp.full((_NL,), rr, jnp.int32)
                    cidx = col + iota
                    picked = plsc.load_gather(buf, [t_vec, rr_vec, cidx])
                    p_val = jnp.exp(picked) / sv
                    keep = p_val <= jnp.float32(_THRESH)
                    obuf[pl.ds(rr * w + col, _NL)] = jnp.where(keep, sv, 1.0)
                    cnt_a = cnt_a + jnp.where(keep, 1.0, 0.0)
                    sp_a = sp_a + jnp.where(keep, picked, 0.0)
                return (cnt_a, sp_a)

            cnt_c, sp_c = lax.fori_loop(0, cpr, body, (cnt_c, sp_c))
            oidx = (bi * sc_rows + qi * rows_tile + r * blk_rows) * w
            opend[r] = pltpu.async_copy(
                obuf, sout_hbm.at[pl.ds(oidx, blk)], osems[par])
        for r in list(opend):
            opend.pop(r).wait()
        accv[pl.ds(0, _NL)] = cnt_c
        accv[pl.ds(_NL, _NL)] = sp_c
        pltpu.sync_copy(accv, acc_hbm.at[pl.ds(wid * 2 * _NL, 2 * _NL)])

    return k(pred, tf)


def _logsum_body(s_ref, out_ref):
    blksum = jnp.sum(jnp.log(s_ref[...]))

    @pl.when(pl.program_id(0) == 0)
    def _():
        out_ref[0, 0] = 0.0

    out_ref[0, 0] += blksum


def _logsum(sprime):
    n = sprime.shape[0]
    rows = 1024
    cols = n // rows
    grid = 2
    return pl.pallas_call(
        _logsum_body,
        grid=(grid,),
        in_specs=[pl.BlockSpec((rows // grid, cols), lambda i: (i, 0))],
        out_specs=pl.BlockSpec(memory_space=pltpu.SMEM),
        out_shape=jax.ShapeDtypeStruct((1, 1), jnp.float32),
    )(sprime.reshape(rows, cols))[0, 0]


def kernel(pred, target):
    b, c, h, w = pred.shape
    n = b * h * w
    k = min(n, _MIN_KEPT)
    thr0 = jnp.full((1, 1), _LOG_THRESH, jnp.float32)
    sc_rows0 = (3 * h) // 16
    hybrid_ok = (
        b <= _NTILES and _NTILES % b == 0
        and sc_rows0 > 0
        and (sc_rows0 % (_NTILES // b)) == 0
        and ((sc_rows0 // (_NTILES // b)) % 4 == 0)
        and w % _NL == 0
        and (b * sc_rows0 * w) % 2048 == 0
    )
    if hybrid_ok:
        rows_tc = h - (3 * h) // 16
        cnt_tc, sum_tc = _stats(pred, target, thr0, log_space=True,
                                rows=rows_tc)
        sprime, accs = _sc_softmax(pred, target, rows_tc)
        ls = _logsum(sprime)
        acc2 = accs.reshape(_NTILES, 2, _NL)
        cnt_sc = jnp.sum(acc2[:, 0, :])
        sp_sc = jnp.sum(acc2[:, 1, :])
        cnt0 = cnt_tc + cnt_sc
        sum0 = sum_tc + ls - sp_sc
    else:
        cnt0, sum0 = _stats(pred, target, thr0, log_space=True)

    def fast():
        return sum0 / jnp.maximum(cnt0, 1.0)

    def slow():
        p = _compute_p(pred, target).reshape(-1)
        kth = _kth_smallest_sc(p, k - 1)
        thr = jnp.maximum(kth, jnp.float32(_THRESH)).reshape(1, 1)
        cnt, ssum = _stats(pred, target, thr, log_space=False)
        return ssum / jnp.maximum(cnt, 1.0)

    return lax.cond(cnt0 >= jnp.float32(k), fast, slow)


# speedup vs baseline: 3.3241x; 1.2239x over previous
import functools

import jax
import jax.numpy as jnp
from jax import lax
from jax.experimental import pallas as pl
from jax.experimental.pallas import tpu as pltpu
from jax.experimental.pallas import tpu_sc as plsc

_THRESH = 0.7
_LOG_THRESH = float(__import__("math").log(0.7))
_MIN_KEPT = 131072

_NC = 2
_NS = 16
_NL = 16
_NTILES = _NC * _NS
_NBINS = 256


def _stats_body(pred_ref, targ_ref, thr_ref, cnt_ref, sum_ref, *, c, log_space):
    t = targ_ref[0]
    s = None
    picked = pred_ref[0, 0]
    for ci in range(c):
        xi = pred_ref[0, ci]
        e = jnp.exp(xi)
        s = e if s is None else s + e
        if ci > 0:
            picked = jnp.where(t == ci, xi, picked)
    logp = picked - jnp.log(s)
    if log_space:
        keep = logp <= thr_ref[0, 0]
    else:
        keep = jnp.exp(logp) <= thr_ref[0, 0]
    cnt_blk = jnp.sum(keep.astype(jnp.float32))
    sum_blk = jnp.sum(jnp.where(keep, -logp, 0.0))

    @pl.when(pl.program_id(0) == 0)
    def _():
        cnt_ref[0, 0] = 0.0
        sum_ref[0, 0] = 0.0

    cnt_ref[0, 0] += cnt_blk
    sum_ref[0, 0] += sum_blk


def _stats(pred, target, thr, log_space, hb=256):
    b, c, h, w = pred.shape
    if h % hb:
        hb = h
    nh = h // hb
    cnt, ssum = pl.pallas_call(
        functools.partial(_stats_body, c=c, log_space=log_space),
        grid=(b * nh,),
        in_specs=[
            pl.BlockSpec((1, c, hb, w), lambda i: (i // nh, 0, i % nh, 0)),
            pl.BlockSpec((1, hb, w), lambda i: (i // nh, i % nh, 0)),
            pl.BlockSpec(memory_space=pltpu.SMEM),
        ],
        out_specs=[
            pl.BlockSpec(memory_space=pltpu.SMEM),
            pl.BlockSpec(memory_space=pltpu.SMEM),
        ],
        out_shape=[
            jax.ShapeDtypeStruct((1, 1), jnp.float32),
            jax.ShapeDtypeStruct((1, 1), jnp.float32),
        ],
    )(pred, target, thr)
    return cnt[0, 0], ssum[0, 0]


def _p_body(pred_ref, targ_ref, p_ref, *, c):
    t = targ_ref[0]
    s = None
    picked = pred_ref[0, 0]
    for ci in range(c):
        xi = pred_ref[0, ci]
        e = jnp.exp(xi)
        s = e if s is None else s + e
        if ci > 0:
            picked = jnp.where(t == ci, xi, picked)
    logp = picked - jnp.log(s)
    p_ref[0] = jnp.exp(logp)


def _compute_p(pred, target, hb=256):
    b, c, h, w = pred.shape
    if h % hb:
        hb = h
    nh = h // hb
    return pl.pallas_call(
        functools.partial(_p_body, c=c),
        grid=(b * nh,),
        in_specs=[
            pl.BlockSpec((1, c, hb, w), lambda i: (i // nh, 0, i % nh, 0)),
            pl.BlockSpec((1, hb, w), lambda i: (i // nh, i % nh, 0)),
        ],
        out_specs=pl.BlockSpec((1, hb, w), lambda i: (i // nh, i % nh, 0)),
        out_shape=jax.ShapeDtypeStruct((b, h, w), jnp.float32),
    )(pred, target)


def _sc_hist(pf, pref16, pass_i):
    n = pf.shape[0]
    chunk = n // _NTILES
    mesh = plsc.VectorSubcoreMesh(
        core_axis_name="c", subcore_axis_name="s",
        num_cores=_NC, num_subcores=_NS,
    )

    @functools.partial(
        pl.kernel,
        mesh=mesh,
        out_type=jax.ShapeDtypeStruct((_NTILES * _NBINS,), jnp.int32),
        compiler_params=pltpu.CompilerParams(needs_layout_passes=False),
        scratch_types=[
            pltpu.VMEM((chunk,), jnp.int32),
            pltpu.VMEM((_NBINS,), jnp.int32),
            pltpu.VMEM((_NL,), jnp.int32),
        ],
    )
    def k(p_hbm, pref_hbm, out_hbm, data_v, hist_v, pref_v):
        wid = lax.axis_index("c") * _NS + lax.axis_index("s")
        pltpu.sync_copy(p_hbm.at[pl.ds(wid * chunk, chunk)], data_v)
        pltpu.sync_copy(pref_hbm, pref_v)
        for j in range(_NBINS // _NL):
            hist_v[pl.ds(j * _NL, _NL)] = jnp.zeros((_NL,), jnp.int32)
        prefv = pref_v[...]
        ones = jnp.ones((_NL,), jnp.int32)

        def body(i, carry):
            bits = data_v[pl.ds(i * _NL, _NL)]
            bin_ = lax.shift_right_logical(bits, 24 - 8 * pass_i) & 255
            if pass_i == 0:
                mask = bits == bits
            else:
                mask = lax.shift_right_logical(bits, 32 - 8 * pass_i) == prefv
            plsc.addupdate_scatter(hist_v, [bin_], ones, mask=mask)
            return carry

        lax.fori_loop(0, chunk // _NL, body, 0)
        pltpu.sync_copy(hist_v, out_hbm.at[pl.ds(wid * _NBINS, _NBINS)])

    return k(pf, pref16)


def _hist_reduce_body(hist_ref, out_ref):
    out_ref[...] = jnp.sum(hist_ref[...], axis=0, keepdims=True)


def _hist_reduce(hist):
    return pl.pallas_call(
        _hist_reduce_body,
        out_shape=jax.ShapeDtypeStruct((1, _NBINS), jnp.int32),
    )(hist.reshape(_NTILES, _NBINS))


def _kth_smallest_sc(pf, rank0):
    bits = lax.bitcast_convert_type(pf, jnp.int32)
    pref = jnp.zeros((), jnp.int32)
    rank = jnp.asarray(rank0, jnp.int32)
    for pass_i in range(4):
        pref16 = jnp.full((_NL,), pref, jnp.int32)
        hist = _sc_hist(bits, pref16, pass_i)
        hsum = _hist_reduce(hist)[0]
        cum = jnp.cumsum(hsum)
        digit = jnp.argmax(cum > rank).astype(jnp.int32)
        below = cum[digit] - hsum[digit]
        rank = rank - below
        pref = pref * 256 + digit
    return lax.bitcast_convert_type(pref, jnp.float32)


def kernel(pred, target):
    b, c, h, w = pred.shape
    n = b * h * w
    k = min(n, _MIN_KEPT)
    thr0 = jnp.full((1, 1), _LOG_THRESH, jnp.float32)
    cnt0, sum0 = _stats(pred, target, thr0, log_space=True)

    def fast():
        return sum0 / jnp.maximum(cnt0, 1.0)

    def slow():
        p = _compute_p(pred, target).reshape(-1)
        kth = _kth_smallest_sc(p, k - 1)
        thr = jnp.maximum(kth, jnp.float32(_THRESH)).reshape(1, 1)
        cnt, ssum = _stats(pred, target, thr, log_space=False)
        return ssum / jnp.maximum(cnt, 1.0)

    return lax.cond(cnt0 >= jnp.float32(k), fast, slow)
